# baseline (speedup 1.0000x reference)
import functools

import jax
import jax.numpy as jnp
from jax import lax
from jax.experimental import pallas as pl
from jax.experimental.pallas import tpu as pltpu
from jax.experimental.pallas import tpu_sc as plsc

N_NODES = 10000
N_EDGES = 160000
HID = 512
NCHUNK = 4
CW = HID // NCHUNK

NC = 2
NS = 16
EDGE_BLK = 96

ROW_BLK = 2000


def _embed_body(x_ref, w_ref, b_ref, *out_refs):
    z = jnp.dot(x_ref[...], w_ref[...], preferred_element_type=jnp.float32)
    z = z + b_ref[...]
    for k in range(NCHUNK):
        out_refs[k][...] = z[:, k * CW:(k + 1) * CW]


def _embed(x, We, be2):
    n, fin = x.shape
    grid = (n // ROW_BLK,)
    return pl.pallas_call(
        _embed_body,
        grid=grid,
        in_specs=[
            pl.BlockSpec((ROW_BLK, fin), lambda i: (i, 0)),
            pl.BlockSpec((fin, HID), lambda i: (0, 0)),
            pl.BlockSpec((1, HID), lambda i: (0, 0)),
        ],
        out_specs=[pl.BlockSpec((ROW_BLK, CW), lambda i: (i, 0))] * NCHUNK,
        out_shape=[jax.ShapeDtypeStruct((n, CW), jnp.float32)] * NCHUNK,
    )(x, We, be2)


NBUF = 3
IDEPTH = 6
EPT = 10368
NBLK = EPT // EDGE_BLK
ACC_ROWS = N_NODES + 16


def _sc_agg_body(*refs):
    h_chunks = refs[0:NCHUNK]
    ei = refs[NCHUNK]
    zeros = refs[NCHUNK + 1]
    out_chunks = refs[NCHUNK + 2:NCHUNK + 2 + NCHUNK]
    rest = refs[NCHUNK + 2 + NCHUNK:]
    ibuf = rest[0:IDEPTH]
    rows = rest[IDEPTH:IDEPTH + NBUF]
    isem = rest[IDEPTH + NBUF:2 * IDEPTH + NBUF]
    gsem = rest[2 * IDEPTH + NBUF:2 * IDEPTH + 2 * NBUF]
    ssem = rest[2 * IDEPTH + 2 * NBUF:2 * IDEPTH + 3 * NBUF]
    acc = rest[2 * IDEPTH + 3 * NBUF]

    c = lax.axis_index("c")
    s = lax.axis_index("s")
    full = 640
    zlast = ACC_ROWS - (NS - 1) * full
    olast = N_NODES - (NS - 1) * full

    def istage(b, k):
        pltpu.make_async_copy(ei.at[s * NBLK + b], ibuf[k], isem[k]).start()

    def iwait(b, k):
        pltpu.make_async_copy(ei.at[s * NBLK + b], ibuf[k], isem[k]).wait()

    def gstart(h_in, j, k):
        pltpu.make_async_copy(h_in.at[ibuf[k].at[0]], rows[j],
                              gsem[j]).start()

    def gwait(h_in, j, k):
        pltpu.make_async_copy(h_in.at[ibuf[k].at[0]], rows[j], gsem[j]).wait()

    for chunk in range(NCHUNK):
        h_in = h_chunks[chunk]
        out = out_chunks[chunk]

        @pl.when(c == chunk // (NCHUNK // NC))
        def _():
            @pl.when(s < NS - 1)
            def _():
                pltpu.sync_copy(zeros, acc.at[pl.ds(s * full, full)])

            @pl.when(s == NS - 1)
            def _():
                pltpu.sync_copy(zeros.at[pl.ds(0, zlast)],
                                acc.at[pl.ds((NS - 1) * full, zlast)])

            plsc.subcore_barrier()

            for k in range(IDEPTH):
                istage(k, k)
            for j in range(2):
                iwait(j, j)
                gstart(h_in, j, j)

            def swait(b, j, u):
                pltpu.make_async_copy(rows[j], acc.at[ibuf[u].at[1]],
                                      ssem[j]).wait()

            def group(g, carry):
                for u in range(IDEPTH):
                    b = g * IDEPTH + u
                    j = u % NBUF
                    jp = (j + 2) % NBUF
                    up = (u + 5) % IDEPTH
                    un = (u + 2) % IDEPTH
                    gwait(h_in, j, u)
                    pltpu.make_async_copy(rows[j], acc.at[ibuf[u].at[1]],
                                          ssem[j]).start(add=True)

                    @pl.when(b >= 1)
                    def _():
                        swait(b - 1, jp, up)

                    @pl.when(jnp.logical_and(b >= 1, b + 5 < NBLK))
                    def _():
                        istage(b + 5, up)

                    @pl.when(b + 2 < NBLK)
                    def _():
                        iwait(b + 2, un)
                        gstart(h_in, jp, un)
                return carry

            lax.fori_loop(0, NBLK // IDEPTH, group, 0)
            swait(NBLK - 1, (NBLK - 1) % NBUF, (NBLK - 1) % IDEPTH)
            plsc.subcore_barrier()

            @pl.when(s < NS - 1)
            def _():
                pltpu.sync_copy(acc.at[pl.ds(s * full, full)],
                                out.at[pl.ds(s * full, full)])

            @pl.when(s == NS - 1)
            def _():
                pltpu.sync_copy(acc.at[pl.ds((NS - 1) * full, olast)],
                                out.at[pl.ds((NS - 1) * full, olast)])

    return None


def _sc_aggregate(h_chunks, ei, zeros):
    mesh = plsc.VectorSubcoreMesh(core_axis_name="c", subcore_axis_name="s",
                                  num_cores=NC, num_subcores=NS)
    kern = pl.kernel(
        _sc_agg_body,
        out_type=[jax.ShapeDtypeStruct((N_NODES, CW), jnp.float32)] * NCHUNK,
        mesh=mesh,
        scratch_types=(
            [pltpu.VMEM((2, EDGE_BLK), jnp.int32)] * IDEPTH
            + [pltpu.VMEM((EDGE_BLK, CW), jnp.float32)] * NBUF
            + [pltpu.SemaphoreType.DMA] * IDEPTH
            + [pltpu.SemaphoreType.DMA] * (2 * NBUF)
            + [pltpu.VMEM_SHARED((ACC_ROWS, CW), jnp.float32)]
        ),
    )
    return kern(*h_chunks, ei, zeros)


def _layer_body(*refs):
    h_refs = refs[0:NCHUNK]
    a_refs = refs[NCHUNK:2 * NCHUNK]
    w1_ref, b1_ref, w2_ref, b2_ref = refs[2 * NCHUNK:2 * NCHUNK + 4]
    out_refs = refs[2 * NCHUNK + 4:]

    h = jnp.concatenate([r[...] for r in h_refs], axis=1)
    agg = jnp.concatenate([r[...] for r in a_refs], axis=1)
    z = h + agg
    z = jnp.dot(z, w1_ref[...], preferred_element_type=jnp.float32) + b1_ref[...]
    z = jnp.maximum(z, 0.0)
    z = jnp.dot(z, w2_ref[...], preferred_element_type=jnp.float32) + b2_ref[...]
    out = h + jnp.maximum(z, 0.0)
    for k in range(NCHUNK):
        out_refs[k][...] = out[:, k * CW:(k + 1) * CW]


def _layer(h_chunks, agg_chunks, W1, b1, W2, b2):
    grid = (N_NODES // ROW_BLK,)
    chunk_spec = pl.BlockSpec((ROW_BLK, CW), lambda i: (i, 0))
    return pl.pallas_call(
        _layer_body,
        grid=grid,
        in_specs=(
            [chunk_spec] * NCHUNK
            + [chunk_spec] * NCHUNK
            + [
                pl.BlockSpec((HID, HID), lambda i: (0, 0)),
                pl.BlockSpec((1, HID), lambda i: (0, 0)),
                pl.BlockSpec((HID, HID), lambda i: (0, 0)),
                pl.BlockSpec((1, HID), lambda i: (0, 0)),
            ]
        ),
        out_specs=[chunk_spec] * NCHUNK,
        out_shape=[jax.ShapeDtypeStruct((N_NODES, CW), jnp.float32)] * NCHUNK,
    )(*h_chunks, *agg_chunks, W1, b1, W2, b2)


def _pool_body(num_graphs, *refs):
    h_refs = refs[0:NCHUNK]
    batch_ref, wr1_ref, br1_ref, wr2_ref, br2_ref = refs[NCHUNK:NCHUNK + 5]
    out_ref = refs[NCHUNK + 5]
    acc_ref = refs[NCHUNK + 6]

    i = pl.program_id(0)

    @pl.when(i == 0)
    def _():
        acc_ref[...] = jnp.zeros_like(acc_ref)

    h = jnp.concatenate([r[...] for r in h_refs], axis=1)
    b = batch_ref[0]
    onehot = (lax.broadcasted_iota(jnp.int32, (num_graphs, ROW_BLK), 0) == b
              ).astype(jnp.float32)
    acc_ref[...] += jnp.dot(onehot, h, preferred_element_type=jnp.float32)

    @pl.when(i == pl.num_programs(0) - 1)
    def _():
        p = acc_ref[...]
        r = jnp.dot(p, wr1_ref[...], preferred_element_type=jnp.float32)
        r = jnp.maximum(r + br1_ref[...], 0.0)
        r = jnp.dot(r, wr2_ref[...], preferred_element_type=jnp.float32)
        out_ref[...] = r + br2_ref[...]


def _pool_readout(h_chunks, batch2, Wr1, br1, Wr2, br2):
    num_graphs = 64
    hid2 = Wr1.shape[1]
    grid = (N_NODES // ROW_BLK,)
    chunk_spec = pl.BlockSpec((ROW_BLK, CW), lambda i: (i, 0))
    return pl.pallas_call(
        functools.partial(_pool_body, num_graphs),
        grid=grid,
        in_specs=(
            [chunk_spec] * NCHUNK
            + [
                pl.BlockSpec((1, 1, ROW_BLK), lambda i: (i, 0, 0)),
                pl.BlockSpec((HID, hid2), lambda i: (0, 0)),
                pl.BlockSpec((1, hid2), lambda i: (0, 0)),
                pl.BlockSpec((hid2, 1), lambda i: (0, 0)),
                pl.BlockSpec((1, 1), lambda i: (0, 0)),
            ]
        ),
        out_specs=pl.BlockSpec((num_graphs, 1), lambda i: (0, 0)),
        out_shape=jax.ShapeDtypeStruct((num_graphs, 1), jnp.float32),
        scratch_shapes=[pltpu.VMEM((num_graphs, HID), jnp.float32)],
    )(*h_chunks, batch2, Wr1, br1, Wr2, br2)


def kernel(h, edge_index, batch, We, be, Wl1, bl1, Wl2, bl2, Wr1, br1, Wr2, br2):
    pad = NS * EPT - N_EDGES
    send = jnp.concatenate(
        [edge_index[0].astype(jnp.int32), jnp.zeros((pad,), jnp.int32)])
    rec = jnp.concatenate(
        [edge_index[1].astype(jnp.int32),
         jnp.full((pad,), N_NODES, jnp.int32)])
    ei = jnp.stack([send.reshape(NS * NBLK, EDGE_BLK),
                    rec.reshape(NS * NBLK, EDGE_BLK)], axis=1)
    batch2 = batch.astype(jnp.int32).reshape(N_NODES // ROW_BLK, 1, ROW_BLK)
    zeros = jnp.zeros((640, CW), jnp.float32)

    h_chunks = _embed(h, We, be.reshape(1, -1))
    for i in range(Wl1.shape[0]):
        agg_chunks = _sc_aggregate(h_chunks, ei, zeros)
        h_chunks = _layer(h_chunks, agg_chunks, Wl1[i], bl1[i].reshape(1, -1),
                          Wl2[i], bl2[i].reshape(1, -1))
    out = _pool_readout(h_chunks, batch2, Wr1, br1.reshape(1, -1),
                        Wr2, br2.reshape(1, -1))
    return out.reshape(-1)


# speedup vs baseline: 2.5533x; 2.5533x over previous
import functools

import jax
import jax.numpy as jnp
from jax import lax
from jax.experimental import pallas as pl
from jax.experimental.pallas import tpu as pltpu
from jax.experimental.pallas import tpu_sc as plsc

N_NODES = 10000
N_EDGES = 160000
HID = 512
NCHUNK = 4
CW = HID // NCHUNK

NC = 2
NS = 16
EDGE_BLK = 80

ROW_BLK = 2000


def _embed_body(x_ref, w_ref, b_ref, *out_refs):
    z = jnp.dot(x_ref[...], w_ref[...], preferred_element_type=jnp.float32)
    z = z + b_ref[...]
    for k in range(NCHUNK):
        out_refs[k][...] = z[:, k * CW:(k + 1) * CW]


def _embed(x, We, be2):
    n, fin = x.shape
    grid = (n // ROW_BLK,)
    return pl.pallas_call(
        _embed_body,
        grid=grid,
        in_specs=[
            pl.BlockSpec((ROW_BLK, fin), lambda i: (i, 0)),
            pl.BlockSpec((fin, HID), lambda i: (0, 0)),
            pl.BlockSpec((1, HID), lambda i: (0, 0)),
        ],
        out_specs=[pl.BlockSpec((ROW_BLK, CW), lambda i: (i, 0))] * NCHUNK,
        out_shape=[jax.ShapeDtypeStruct((n, CW), jnp.float32)] * NCHUNK,
    )(x, We, be2)


NBUF = 3
IDEPTH = 6
EPT = 10080
NBLK = EPT // EDGE_BLK
PAD_T = EPT - N_EDGES // NS
ACC_ROWS = N_NODES + PAD_T


def _sc_agg_body(*refs):
    h_chunks = refs[0:NCHUNK]
    ei = refs[NCHUNK]
    zeros = refs[NCHUNK + 1]
    out_chunks = refs[NCHUNK + 2:NCHUNK + 2 + NCHUNK]
    rest = refs[NCHUNK + 2 + NCHUNK:]
    ibuf = rest[0:IDEPTH]
    rows = rest[IDEPTH:IDEPTH + NBUF]
    isem = rest[IDEPTH + NBUF:2 * IDEPTH + NBUF]
    gsem = rest[2 * IDEPTH + NBUF:2 * IDEPTH + 2 * NBUF]
    ssem = rest[2 * IDEPTH + 2 * NBUF:2 * IDEPTH + 3 * NBUF]
    acc = rest[2 * IDEPTH + 3 * NBUF]

    c = lax.axis_index("c")
    s = lax.axis_index("s")
    full = 640
    zlast = ACC_ROWS - (NS - 1) * full
    olast = N_NODES - (NS - 1) * full

    def istage(b, k):
        pltpu.make_async_copy(ei.at[s * NBLK + b], ibuf[k], isem[k]).start()

    def iwait(b, k):
        pltpu.make_async_copy(ei.at[s * NBLK + b], ibuf[k], isem[k]).wait()

    def gstart(h_in, j, k):
        pltpu.make_async_copy(h_in.at[ibuf[k].at[0]], rows[j],
                              gsem[j]).start()

    def gwait(h_in, j, k):
        pltpu.make_async_copy(h_in.at[ibuf[k].at[0]], rows[j], gsem[j]).wait()

    for chunk in range(NCHUNK):
        h_in = h_chunks[chunk]
        out = out_chunks[chunk]

        @pl.when(c == chunk // (NCHUNK // NC))
        def _():
            @pl.when(s < NS - 1)
            def _():
                pltpu.sync_copy(zeros, acc.at[pl.ds(s * full, full)])

            @pl.when(s == NS - 1)
            def _():
                pltpu.sync_copy(zeros.at[pl.ds(0, zlast)],
                                acc.at[pl.ds((NS - 1) * full, zlast)])

            plsc.subcore_barrier()

            for k in range(IDEPTH):
                istage(k, k)
            for j in range(2):
                iwait(j, j)
                gstart(h_in, j, j)

            def swait(b, j, u):
                pltpu.make_async_copy(rows[j], acc.at[ibuf[u].at[1]],
                                      ssem[j]).wait()

            def group(g, carry):
                for u in range(IDEPTH):
                    b = g * IDEPTH + u
                    j = u % NBUF
                    jp = (j + 2) % NBUF
                    up = (u + 5) % IDEPTH
                    un = (u + 2) % IDEPTH
                    gwait(h_in, j, u)
                    pltpu.make_async_copy(rows[j], acc.at[ibuf[u].at[1]],
                                          ssem[j]).start(add=True)

                    @pl.when(b >= 1)
                    def _():
                        swait(b - 1, jp, up)

                    @pl.when(jnp.logical_and(b >= 1, b + 5 < NBLK))
                    def _():
                        istage(b + 5, up)

                    @pl.when(b + 2 < NBLK)
                    def _():
                        iwait(b + 2, un)
                        gstart(h_in, jp, un)
                return carry

            lax.fori_loop(0, NBLK // IDEPTH, group, 0)
            swait(NBLK - 1, (NBLK - 1) % NBUF, (NBLK - 1) % IDEPTH)
            plsc.subcore_barrier()

            @pl.when(s < NS - 1)
            def _():
                pltpu.sync_copy(acc.at[pl.ds(s * full, full)],
                                out.at[pl.ds(s * full, full)])

            @pl.when(s == NS - 1)
            def _():
                pltpu.sync_copy(acc.at[pl.ds((NS - 1) * full, olast)],
                                out.at[pl.ds((NS - 1) * full, olast)])

    return None


def _sc_aggregate(h_chunks, ei, zeros):
    mesh = plsc.VectorSubcoreMesh(core_axis_name="c", subcore_axis_name="s",
                                  num_cores=NC, num_subcores=NS)
    kern = pl.kernel(
        _sc_agg_body,
        out_type=[jax.ShapeDtypeStruct((N_NODES, CW), jnp.float32)] * NCHUNK,
        mesh=mesh,
        scratch_types=(
            [pltpu.VMEM((2, EDGE_BLK), jnp.int32)] * IDEPTH
            + [pltpu.VMEM((EDGE_BLK, CW), jnp.float32)] * NBUF
            + [pltpu.SemaphoreType.DMA] * IDEPTH
            + [pltpu.SemaphoreType.DMA] * (2 * NBUF)
            + [pltpu.VMEM_SHARED((ACC_ROWS, CW), jnp.float32)]
        ),
    )
    return kern(*h_chunks, ei, zeros)


def _layer_body(*refs):
    h_refs = refs[0:NCHUNK]
    a_refs = refs[NCHUNK:2 * NCHUNK]
    w1_ref, b1_ref, w2_ref, b2_ref = refs[2 * NCHUNK:2 * NCHUNK + 4]
    out_refs = refs[2 * NCHUNK + 4:]

    h = jnp.concatenate([r[...] for r in h_refs], axis=1)
    agg = jnp.concatenate([r[...] for r in a_refs], axis=1)
    z = h + agg
    z = jnp.dot(z, w1_ref[...], preferred_element_type=jnp.float32) + b1_ref[...]
    z = jnp.maximum(z, 0.0)
    z = jnp.dot(z, w2_ref[...], preferred_element_type=jnp.float32) + b2_ref[...]
    out = h + jnp.maximum(z, 0.0)
    for k in range(NCHUNK):
        out_refs[k][...] = out[:, k * CW:(k + 1) * CW]


def _layer(h_chunks, agg_chunks, W1, b1, W2, b2):
    grid = (N_NODES // ROW_BLK,)
    chunk_spec = pl.BlockSpec((ROW_BLK, CW), lambda i: (i, 0))
    return pl.pallas_call(
        _layer_body,
        grid=grid,
        in_specs=(
            [chunk_spec] * NCHUNK
            + [chunk_spec] * NCHUNK
            + [
                pl.BlockSpec((HID, HID), lambda i: (0, 0)),
                pl.BlockSpec((1, HID), lambda i: (0, 0)),
                pl.BlockSpec((HID, HID), lambda i: (0, 0)),
                pl.BlockSpec((1, HID), lambda i: (0, 0)),
            ]
        ),
        out_specs=[chunk_spec] * NCHUNK,
        out_shape=[jax.ShapeDtypeStruct((N_NODES, CW), jnp.float32)] * NCHUNK,
    )(*h_chunks, *agg_chunks, W1, b1, W2, b2)


def _pool_body(num_graphs, *refs):
    h_refs = refs[0:NCHUNK]
    batch_ref, wr1_ref, br1_ref, wr2_ref, br2_ref = refs[NCHUNK:NCHUNK + 5]
    out_ref = refs[NCHUNK + 5]
    acc_ref = refs[NCHUNK + 6]

    i = pl.program_id(0)

    @pl.when(i == 0)
    def _():
        acc_ref[...] = jnp.zeros_like(acc_ref)

    h = jnp.concatenate([r[...] for r in h_refs], axis=1)
    b = batch_ref[0]
    onehot = (lax.broadcasted_iota(jnp.int32, (num_graphs, ROW_BLK), 0) == b
              ).astype(jnp.float32)
    acc_ref[...] += jnp.dot(onehot, h, preferred_element_type=jnp.float32)

    @pl.when(i == pl.num_programs(0) - 1)
    def _():
        p = acc_ref[...]
        r = jnp.dot(p, wr1_ref[...], preferred_element_type=jnp.float32)
        r = jnp.maximum(r + br1_ref[...], 0.0)
        r = jnp.dot(r, wr2_ref[...], preferred_element_type=jnp.float32)
        out_ref[...] = r + br2_ref[...]


def _pool_readout(h_chunks, batch2, Wr1, br1, Wr2, br2):
    num_graphs = 64
    hid2 = Wr1.shape[1]
    grid = (N_NODES // ROW_BLK,)
    chunk_spec = pl.BlockSpec((ROW_BLK, CW), lambda i: (i, 0))
    return pl.pallas_call(
        functools.partial(_pool_body, num_graphs),
        grid=grid,
        in_specs=(
            [chunk_spec] * NCHUNK
            + [
                pl.BlockSpec((1, 1, ROW_BLK), lambda i: (i, 0, 0)),
                pl.BlockSpec((HID, hid2), lambda i: (0, 0)),
                pl.BlockSpec((1, hid2), lambda i: (0, 0)),
                pl.BlockSpec((hid2, 1), lambda i: (0, 0)),
                pl.BlockSpec((1, 1), lambda i: (0, 0)),
            ]
        ),
        out_specs=pl.BlockSpec((num_graphs, 1), lambda i: (0, 0)),
        out_shape=jax.ShapeDtypeStruct((num_graphs, 1), jnp.float32),
        scratch_shapes=[pltpu.VMEM((num_graphs, HID), jnp.float32)],
    )(*h_chunks, batch2, Wr1, br1, Wr2, br2)


def kernel(h, edge_index, batch, We, be, Wl1, bl1, Wl2, bl2, Wr1, br1, Wr2, br2):
    ept_real = N_EDGES // NS
    send = jnp.concatenate(
        [edge_index[0].astype(jnp.int32).reshape(NS, ept_real),
         jnp.zeros((NS, PAD_T), jnp.int32)], axis=1)
    rec = jnp.concatenate(
        [edge_index[1].astype(jnp.int32).reshape(NS, ept_real),
         jnp.broadcast_to(N_NODES + jnp.arange(PAD_T, dtype=jnp.int32),
                          (NS, PAD_T))], axis=1)
    ei = jnp.stack([send.reshape(NS * NBLK, EDGE_BLK),
                    rec.reshape(NS * NBLK, EDGE_BLK)], axis=1)
    batch2 = batch.astype(jnp.int32).reshape(N_NODES // ROW_BLK, 1, ROW_BLK)
    zeros = jnp.zeros((640, CW), jnp.float32)

    h_chunks = _embed(h, We, be.reshape(1, -1))
    for i in range(Wl1.shape[0]):
        agg_chunks = _sc_aggregate(h_chunks, ei, zeros)
        h_chunks = _layer(h_chunks, agg_chunks, Wl1[i], bl1[i].reshape(1, -1),
                          Wl2[i], bl2[i].reshape(1, -1))
    out = _pool_readout(h_chunks, batch2, Wr1, br1.reshape(1, -1),
                        Wr2, br2.reshape(1, -1))
    return out.reshape(-1)
